# baseline (speedup 1.0000x reference)
import functools

import jax
import jax.numpy as jnp
from jax import lax
from jax.experimental import pallas as pl
from jax.experimental.pallas import tpu as pltpu
from jax.experimental.pallas import tpu_sc as plsc

E = 160000
T = 640000
D = 256

_BE = 1600
_BT = 2000


def _relu(v):
    return jnp.maximum(v, 0.0)


def _front_body(x_ref, rbf_ref, wji_ref, bji_ref, wkj_ref, bkj_ref,
                wrbf_ref, brbf_ref, xji_ref, g_ref):
    xb = x_ref[...]
    xji_ref[...] = _relu(
        jnp.dot(xb, wji_ref[...], preferred_element_type=jnp.float32)
        + bji_ref[...])
    xkj = _relu(
        jnp.dot(xb, wkj_ref[...], preferred_element_type=jnp.float32)
        + bkj_ref[...])
    rt = _relu(
        jnp.dot(rbf_ref[...], wrbf_ref[...], preferred_element_type=jnp.float32)
        + brbf_ref[...])
    g_ref[:, :D] = xkj
    g_ref[:, D:] = rt * xkj


def _front(x, rbf8, w_ji, b_ji, w_kj, b_kj, w_rbf, b_rbf):
    nblk = E // _BE
    return pl.pallas_call(
        _front_body,
        grid=(nblk,),
        in_specs=[
            pl.BlockSpec((_BE, D), lambda i: (i, 0)),
            pl.BlockSpec((_BE, 8), lambda i: (i, 0)),
            pl.BlockSpec((D, D), lambda i: (0, 0)),
            pl.BlockSpec((1, D), lambda i: (0, 0)),
            pl.BlockSpec((D, D), lambda i: (0, 0)),
            pl.BlockSpec((1, D), lambda i: (0, 0)),
            pl.BlockSpec((8, D), lambda i: (0, 0)),
            pl.BlockSpec((1, D), lambda i: (0, 0)),
        ],
        out_specs=[
            pl.BlockSpec((_BE, D), lambda i: (i, 0)),
            pl.BlockSpec((_BE, 2 * D), lambda i: (i, 0)),
        ],
        out_shape=[
            jax.ShapeDtypeStruct((E, D), jnp.float32),
            jax.ShapeDtypeStruct((E, 2 * D), jnp.float32),
        ],
        compiler_params=pltpu.CompilerParams(
            dimension_semantics=("arbitrary",)),
    )(x, rbf8, w_ji, b_ji.reshape(1, D), w_kj, b_kj.reshape(1, D),
      w_rbf, b_rbf.reshape(1, D))


def _cos_body(cos_ref, w_ref, b_ref, out_ref):
    out_ref[...] = _relu(
        jnp.dot(cos_ref[...], w_ref[...], preferred_element_type=jnp.float32)
        + b_ref[...])


def _cos_t(cos8, w_cos, b_cos):
    nblk = cos8.shape[0] // _BT
    return pl.pallas_call(
        _cos_body,
        grid=(nblk,),
        in_specs=[
            pl.BlockSpec((_BT, 8), lambda i: (i, 0)),
            pl.BlockSpec((8, D), lambda i: (0, 0)),
            pl.BlockSpec((1, D), lambda i: (0, 0)),
        ],
        out_specs=pl.BlockSpec((_BT, D), lambda i: (i, 0)),
        out_shape=jax.ShapeDtypeStruct((cos8.shape[0], D), jnp.float32),
        compiler_params=pltpu.CompilerParams(
            dimension_semantics=("arbitrary",)),
    )(cos8, w_cos, b_cos.reshape(1, D))


def _mlp_body(xji_ref, conv_ref, conv2_ref, x_ref, w1, b1, w2, b2, w3, b3,
              w4, b4, w5, b5, w6, b6, w7, b7, cx_ref, cf_ref, out_ref):
    def ff(v, w, b):
        return _relu(
            jnp.dot(v, w[...], preferred_element_type=jnp.float32) + b[...])

    x2 = xji_ref[...] + (conv_ref[...] + conv2_ref[...])
    x2 = ff(ff(x2, w1, b1), w2, b2) + x2
    xo = ff(x2, w3, b3) + cx_ref[0, 0] * x_ref[...]
    xo = ff(ff(xo, w4, b4), w5, b5) + xo
    xo = ff(ff(xo, w6, b6), w7, b7) + xo
    out_ref[...] = xo * cf_ref[0, 0]


def _mlp(x_ji, conv, conv2, x, ws_bs, coef_x, coef_final):
    nblk = E // _BE
    dspec = pl.BlockSpec((_BE, D), lambda i: (i, 0))
    wspec = pl.BlockSpec((D, D), lambda i: (0, 0))
    bspec = pl.BlockSpec((1, D), lambda i: (0, 0))
    sspec = pl.BlockSpec(memory_space=pltpu.SMEM)
    in_specs = [dspec, dspec, dspec, dspec]
    args = [x_ji, conv, conv2, x]
    for w, b in ws_bs:
        in_specs += [wspec, bspec]
        args += [w, b.reshape(1, D)]
    in_specs += [sspec, sspec]
    args += [coef_x.reshape(1, 1), coef_final.reshape(1, 1)]
    return pl.pallas_call(
        _mlp_body,
        grid=(nblk,),
        in_specs=in_specs,
        out_specs=pl.BlockSpec((_BE, D), lambda i: (i, 0)),
        out_shape=jax.ShapeDtypeStruct((E, D), jnp.float32),
        compiler_params=pltpu.CompilerParams(
            dimension_semantics=("arbitrary",)),
    )(*args)


_NW = 32
_LANES = 16
_NSPLIT = 4
_TH = T // _NSPLIT
_TS = _TH // _NW
_BB = 40
_NBATCH = _TS // _BB
_KCH = 1000
_BPC = _KCH // _BB
_NSLOT = 3


def _gather_body(g_hbm, kj_hbm, out_hbm,
                 kjb, g0, g1, g2, sg0, sg1, sg2, sv0, sv1, sv2):
    c = lax.axis_index("c")
    s = lax.axis_index("s")
    wid = s * 2 + c
    tbase = wid * _TS
    gs = (g0, g1, g2)
    sgs = (sg0, sg1, sg2)
    svs = (sv0, sv1, sv2)

    def issue(b, slot):
        @pl.when(lax.rem(b, _BPC) == 0)
        def _():
            ch = b // _BPC
            pltpu.sync_copy(
                kj_hbm.at[pl.ds(tbase + ch * _KCH, _KCH)],
                kjb.at[pl.ds(lax.rem(ch, 2) * _KCH, _KCH)])
        ch = b // _BPC
        loc = lax.rem(ch, 2) * _KCH + lax.rem(b, _BPC) * _BB
        @pl.when(b >= _NSLOT)
        def _():
            pltpu.make_async_copy(
                gs[slot], out_hbm.at[pl.ds(0, _BB)], svs[slot]).wait()
        pltpu.async_copy(
            g_hbm.at[kjb.at[pl.ds(loc, _BB)]], gs[slot], sgs[slot])

    def process(b, par):
        pltpu.make_async_copy(
            g_hbm.at[pl.ds(0, _BB)], gs[par], sgs[par]).wait()
        pltpu.async_copy(
            gs[par], out_hbm.at[pl.ds(tbase + b * _BB, _BB)], svs[par])

        @pl.when(b + 2 < _NBATCH)
        def _():
            issue(b + 2, (par + 2) % _NSLOT)

    for par in range(2):
        issue(jnp.int32(par), par)

    def group_body(q, _):
        for par in range(_NSLOT):
            process(_NSLOT * q + par, par)
        return 0

    lax.fori_loop(0, _NBATCH // _NSLOT, group_body, 0)

    for b in range(_NBATCH - _NBATCH % _NSLOT, _NBATCH):
        process(jnp.int32(b), b % _NSLOT)

    for par in range(_NSLOT):
        pltpu.make_async_copy(
            gs[par], out_hbm.at[pl.ds(0, _BB)], svs[par]).wait()


def _gather_half(G, kj_half):
    f = functools.partial(
        pl.kernel,
        mesh=plsc.VectorSubcoreMesh(core_axis_name="c", subcore_axis_name="s"),
        out_type=jax.ShapeDtypeStruct((_TH, 2 * D), jnp.float32),
        scratch_types=[
            pltpu.VMEM((2 * _KCH,), jnp.int32),
            pltpu.VMEM((_BB, 2 * D), jnp.float32),
            pltpu.VMEM((_BB, 2 * D), jnp.float32),
            pltpu.VMEM((_BB, 2 * D), jnp.float32),
            pltpu.SemaphoreType.DMA,
            pltpu.SemaphoreType.DMA,
            pltpu.SemaphoreType.DMA,
            pltpu.SemaphoreType.DMA,
            pltpu.SemaphoreType.DMA,
            pltpu.SemaphoreType.DMA,
        ],
    )(_gather_body)
    return f(G, kj_half)


def _combine_body(g_ref, cos_ref, out_ref):
    g = g_ref[...]
    out_ref[...] = g[:, D:] + cos_ref[...] * g[:, :D]


def _combine(gathered, cos_half):
    nblk = _TH // _BT
    return pl.pallas_call(
        _combine_body,
        grid=(nblk,),
        in_specs=[
            pl.BlockSpec((_BT, 2 * D), lambda i: (i, 0)),
            pl.BlockSpec((_BT, D), lambda i: (i, 0)),
        ],
        out_specs=pl.BlockSpec((_BT, D), lambda i: (i, 0)),
        out_shape=jax.ShapeDtypeStruct((_TH, D), jnp.float32),
        compiler_params=pltpu.CompilerParams(
            dimension_semantics=("arbitrary",)),
    )(gathered, cos_half)


def kernel(x, rbf, cos_ijk, id_expand_kj, id_reduce_ji,
           W_ji, b_ji, W_kj, b_kj, W_rbf, b_rbf, W_cos, b_cos,
           W1, b1, W2, b2, W3, b3, W4, b4, W5, b5, W6, b6, W7, b7,
           coef_rbf_a, coef_cos_a, coef_rbf_b, coef_cos_b, coef_x,
           coef_final):
    kj = id_expand_kj.astype(jnp.int32)
    ji = id_reduce_ji.astype(jnp.int32)

    ca_rbf = coef_rbf_a[0] * coef_rbf_b[0]
    cb_rbf = coef_rbf_b[0]
    ca_cos = coef_cos_a[0] * coef_cos_b[0]
    cb_cos = coef_cos_b[0]

    rbf8 = jnp.pad(rbf, ((0, 0), (0, 2)))
    cos8 = jnp.pad(cos_ijk, ((0, 0), (0, 1)))
    w_rbf8 = jnp.pad(ca_rbf * W_rbf, ((0, 2), (0, 0)))
    w_cos8 = jnp.pad(ca_cos * W_cos, ((0, 1), (0, 0)))

    x_ji, G = _front(x, rbf8, W_ji, b_ji, W_kj, b_kj, w_rbf8, cb_rbf * b_rbf)

    conv_halves = []
    for h in range(_NSPLIT):
        sl = slice(h * _TH, (h + 1) * _TH)
        cos_h = _cos_t(cos8[sl], w_cos8, cb_cos * b_cos)
        gath = _gather_half(G, kj[sl])
        v = _combine(gath, cos_h)
        conv_halves.append(jax.ops.segment_sum(v, ji[sl], num_segments=E))

    ws_bs = [(W1, b1), (W2, b2), (W3, b3), (W4, b4), (W5, b5), (W6, b6),
             (W7, b7)]
    conv_a = conv_halves[0] + conv_halves[1]
    conv_b = conv_halves[2] + conv_halves[3]
    return _mlp(x_ji, conv_a, conv_b, x, ws_bs, coef_x, coef_final)


# speedup vs baseline: 1.1159x; 1.1159x over previous
import functools

import jax
import jax.numpy as jnp
from jax import lax
from jax.experimental import pallas as pl
from jax.experimental.pallas import tpu as pltpu
from jax.experimental.pallas import tpu_sc as plsc

E = 160000
T = 640000
D = 256

_BE = 1600
_BT = 2560


def _relu(v):
    return jnp.maximum(v, 0.0)


def _front_body(x_ref, rbf_ref, wji_ref, bji_ref, wkj_ref, bkj_ref,
                wrbf_ref, brbf_ref, xji_ref, g_ref):
    xb = x_ref[...]
    xji_ref[...] = _relu(
        jnp.dot(xb, wji_ref[...], preferred_element_type=jnp.float32)
        + bji_ref[...])
    xkj = _relu(
        jnp.dot(xb, wkj_ref[...], preferred_element_type=jnp.float32)
        + bkj_ref[...])
    rt = _relu(
        jnp.dot(rbf_ref[...], wrbf_ref[...], preferred_element_type=jnp.float32)
        + brbf_ref[...])
    g_ref[:, :D] = xkj
    g_ref[:, D:] = rt * xkj


def _front(x, rbf8, w_ji, b_ji, w_kj, b_kj, w_rbf, b_rbf):
    nblk = E // _BE
    return pl.pallas_call(
        _front_body,
        grid=(nblk,),
        in_specs=[
            pl.BlockSpec((_BE, D), lambda i: (i, 0)),
            pl.BlockSpec((_BE, 8), lambda i: (i, 0)),
            pl.BlockSpec((D, D), lambda i: (0, 0)),
            pl.BlockSpec((1, D), lambda i: (0, 0)),
            pl.BlockSpec((D, D), lambda i: (0, 0)),
            pl.BlockSpec((1, D), lambda i: (0, 0)),
            pl.BlockSpec((8, D), lambda i: (0, 0)),
            pl.BlockSpec((1, D), lambda i: (0, 0)),
        ],
        out_specs=[
            pl.BlockSpec((_BE, D), lambda i: (i, 0)),
            pl.BlockSpec((_BE, 2 * D), lambda i: (i, 0)),
        ],
        out_shape=[
            jax.ShapeDtypeStruct((E, D), jnp.float32),
            jax.ShapeDtypeStruct((E, 2 * D), jnp.float32),
        ],
        compiler_params=pltpu.CompilerParams(
            dimension_semantics=("arbitrary",)),
    )(x, rbf8, w_ji, b_ji.reshape(1, D), w_kj, b_kj.reshape(1, D),
      w_rbf, b_rbf.reshape(1, D))


def _cos_body(cos_ref, w_ref, b_ref, out_ref):
    out_ref[...] = _relu(
        jnp.dot(cos_ref[...], w_ref[...], preferred_element_type=jnp.float32)
        + b_ref[...])


def _cos_t(cos8, w_cos, b_cos):
    nblk = cos8.shape[0] // _BT
    return pl.pallas_call(
        _cos_body,
        grid=(nblk,),
        in_specs=[
            pl.BlockSpec((_BT, 8), lambda i: (i, 0)),
            pl.BlockSpec((8, D), lambda i: (0, 0)),
            pl.BlockSpec((1, D), lambda i: (0, 0)),
        ],
        out_specs=pl.BlockSpec((_BT, D), lambda i: (i, 0)),
        out_shape=jax.ShapeDtypeStruct((T, D), jnp.float32),
        compiler_params=pltpu.CompilerParams(
            dimension_semantics=("arbitrary",)),
    )(cos8, w_cos, b_cos.reshape(1, D))


def _mlp_body(xji_ref, conv_ref, conv2_ref, x_ref, w1, b1, w2, b2, w3, b3,
              w4, b4, w5, b5, w6, b6, w7, b7, cx_ref, cf_ref, out_ref):
    def ff(v, w, b):
        return _relu(
            jnp.dot(v, w[...], preferred_element_type=jnp.float32) + b[...])

    x2 = xji_ref[...] + (conv_ref[...] + conv2_ref[...])
    x2 = ff(ff(x2, w1, b1), w2, b2) + x2
    xo = ff(x2, w3, b3) + cx_ref[0, 0] * x_ref[...]
    xo = ff(ff(xo, w4, b4), w5, b5) + xo
    xo = ff(ff(xo, w6, b6), w7, b7) + xo
    out_ref[...] = xo * cf_ref[0, 0]


def _mlp(x_ji, conv, conv2, x, ws_bs, coef_x, coef_final):
    nblk = E // _BE
    dspec = pl.BlockSpec((_BE, D), lambda i: (i, 0))
    wspec = pl.BlockSpec((D, D), lambda i: (0, 0))
    bspec = pl.BlockSpec((1, D), lambda i: (0, 0))
    sspec = pl.BlockSpec(memory_space=pltpu.SMEM)
    in_specs = [dspec, dspec, dspec, dspec]
    args = [x_ji, conv, conv2, x]
    for w, b in ws_bs:
        in_specs += [wspec, bspec]
        args += [w, b.reshape(1, D)]
    in_specs += [sspec, sspec]
    args += [coef_x.reshape(1, 1), coef_final.reshape(1, 1)]
    return pl.pallas_call(
        _mlp_body,
        grid=(nblk,),
        in_specs=in_specs,
        out_specs=pl.BlockSpec((_BE, D), lambda i: (i, 0)),
        out_shape=jax.ShapeDtypeStruct((E, D), jnp.float32),
        compiler_params=pltpu.CompilerParams(
            dimension_semantics=("arbitrary",)),
    )(*args)


_NW = 32
_LANES = 16
_TH = T // 2
_TS = _TH // _NW
_BB = 40
_NBATCH = _TS // _BB
_KCH = 2000
_BPC = _KCH // _BB
_NSLOT = 3


def _gather_body(g_hbm, kj_hbm, out_hbm,
                 kjb, g0, g1, g2, sg0, sg1, sg2, sv0, sv1, sv2):
    c = lax.axis_index("c")
    s = lax.axis_index("s")
    wid = s * 2 + c
    tbase = wid * _TS
    gs = (g0, g1, g2)
    sgs = (sg0, sg1, sg2)
    svs = (sv0, sv1, sv2)

    def issue(b, slot):
        @pl.when(lax.rem(b, _BPC) == 0)
        def _():
            ch = b // _BPC
            pltpu.sync_copy(
                kj_hbm.at[pl.ds(tbase + ch * _KCH, _KCH)],
                kjb.at[pl.ds(lax.rem(ch, 2) * _KCH, _KCH)])
        ch = b // _BPC
        loc = lax.rem(ch, 2) * _KCH + lax.rem(b, _BPC) * _BB
        @pl.when(b >= _NSLOT)
        def _():
            pltpu.make_async_copy(
                gs[slot], out_hbm.at[pl.ds(0, _BB)], svs[slot]).wait()
        pltpu.async_copy(
            g_hbm.at[kjb.at[pl.ds(loc, _BB)]], gs[slot], sgs[slot])

    def process(b, par):
        pltpu.make_async_copy(
            g_hbm.at[pl.ds(0, _BB)], gs[par], sgs[par]).wait()
        pltpu.async_copy(
            gs[par], out_hbm.at[pl.ds(tbase + b * _BB, _BB)], svs[par])

        @pl.when(b + 2 < _NBATCH)
        def _():
            issue(b + 2, (par + 2) % _NSLOT)

    for par in range(2):
        issue(jnp.int32(par), par)

    def group_body(q, _):
        for par in range(_NSLOT):
            process(_NSLOT * q + par, par)
        return 0

    lax.fori_loop(0, _NBATCH // _NSLOT, group_body, 0)

    for b in range(_NBATCH - _NBATCH % _NSLOT, _NBATCH):
        process(jnp.int32(b), b % _NSLOT)

    for par in range(_NSLOT):
        pltpu.make_async_copy(
            gs[par], out_hbm.at[pl.ds(0, _BB)], svs[par]).wait()


def _gather_half(G, kj_half):
    f = functools.partial(
        pl.kernel,
        mesh=plsc.VectorSubcoreMesh(core_axis_name="c", subcore_axis_name="s"),
        out_type=jax.ShapeDtypeStruct((_TH, 2 * D), jnp.float32),
        scratch_types=[
            pltpu.VMEM((2 * _KCH,), jnp.int32),
            pltpu.VMEM((_BB, 2 * D), jnp.float32),
            pltpu.VMEM((_BB, 2 * D), jnp.float32),
            pltpu.VMEM((_BB, 2 * D), jnp.float32),
            pltpu.SemaphoreType.DMA,
            pltpu.SemaphoreType.DMA,
            pltpu.SemaphoreType.DMA,
            pltpu.SemaphoreType.DMA,
            pltpu.SemaphoreType.DMA,
            pltpu.SemaphoreType.DMA,
        ],
    )(_gather_body)
    return f(G, kj_half)


def _combine_body(g_ref, cos_ref, out_ref):
    g = g_ref[...]
    out_ref[...] = g[:, D:] + cos_ref[...] * g[:, :D]


def _combine(gathered, cos_half):
    nblk = _TH // _BT
    return pl.pallas_call(
        _combine_body,
        grid=(nblk,),
        in_specs=[
            pl.BlockSpec((_BT, 2 * D), lambda i: (i, 0)),
            pl.BlockSpec((_BT, D), lambda i: (i, 0)),
        ],
        out_specs=pl.BlockSpec((_BT, D), lambda i: (i, 0)),
        out_shape=jax.ShapeDtypeStruct((_TH, D), jnp.float32),
        compiler_params=pltpu.CompilerParams(
            dimension_semantics=("arbitrary",)),
    )(gathered, cos_half)


def kernel(x, rbf, cos_ijk, id_expand_kj, id_reduce_ji,
           W_ji, b_ji, W_kj, b_kj, W_rbf, b_rbf, W_cos, b_cos,
           W1, b1, W2, b2, W3, b3, W4, b4, W5, b5, W6, b6, W7, b7,
           coef_rbf_a, coef_cos_a, coef_rbf_b, coef_cos_b, coef_x,
           coef_final):
    kj = id_expand_kj.astype(jnp.int32)
    ji = id_reduce_ji.astype(jnp.int32)

    ca_rbf = coef_rbf_a[0] * coef_rbf_b[0]
    cb_rbf = coef_rbf_b[0]
    ca_cos = coef_cos_a[0] * coef_cos_b[0]
    cb_cos = coef_cos_b[0]

    rbf8 = jnp.pad(rbf, ((0, 0), (0, 2)))
    cos8 = jnp.pad(cos_ijk, ((0, 0), (0, 1)))
    w_rbf8 = jnp.pad(ca_rbf * W_rbf, ((0, 2), (0, 0)))
    w_cos8 = jnp.pad(ca_cos * W_cos, ((0, 1), (0, 0)))

    x_ji, G = _front(x, rbf8, W_ji, b_ji, W_kj, b_kj, w_rbf8, cb_rbf * b_rbf)

    conv_halves = []
    for h in range(2):
        sl = slice(h * _TH, (h + 1) * _TH)
        cos_h = _cos_t(cos8[sl], w_cos8, cb_cos * b_cos)
        gath = _gather_half(G, kj[sl])
        v = _combine(gath, cos_h)
        conv_halves.append(jax.ops.segment_sum(v, ji[sl], num_segments=E))

    ws_bs = [(W1, b1), (W2, b2), (W3, b3), (W4, b4), (W5, b5), (W6, b6),
             (W7, b7)]
    return _mlp(x_ji, conv_halves[0], conv_halves[1], x, ws_bs,
                coef_x, coef_final)
